# Initial kernel scaffold; baseline (speedup 1.0000x reference)
import jax
import jax.numpy as jnp
from jax.experimental import pallas as pl


def kernel(actions0, step_embs, h0, c0, nbr, node_pre_embedding, W1_0, b1_0, W1_1, b1_1, W2_0, b2_0, W2_1, b2_1, W3_0, b3_0, W_ih, W_hh, b_ih, b_hh):
    raise NotImplementedError("write your pallas kernel here")


# speedup vs baseline: 2.6054x; 2.6054x over previous
import functools

import jax
import jax.numpy as jnp
import numpy as np
from jax import lax
from jax.experimental import pallas as pl
from jax.experimental.pallas import tpu as pltpu
from jax.experimental.pallas import tpu_sc as plsc

_N = 50000
_DEG = 32
_DF = 128
_B = 4096
_H = 256
_DELAYED = 100.0

_NC = 2
_NS = 16
_NW = _NC * _NS
_APW = _B // _NW
_ROWS = _B * _DEG
_RPW = _ROWS // _NW
_CH = 128
_NCH = _RPW // _CH

_BB = 128
_GRID = _B // _BB


def _sc_gather_actions(actions, nbr, emb):
    mesh = plsc.VectorSubcoreMesh(
        core_axis_name="c", subcore_axis_name="s",
        num_cores=_NC, num_subcores=_NS)

    @functools.partial(
        pl.kernel,
        out_type=(jax.ShapeDtypeStruct((_B, _DEG), jnp.int32),
                  jax.ShapeDtypeStruct((_B, _DF), jnp.float32)),
        mesh=mesh,
        scratch_types=[
            pltpu.VMEM((_APW,), jnp.int32),
            pltpu.VMEM((_APW, _DEG), jnp.int32),
            pltpu.VMEM((_APW, _DF), jnp.float32),
            pltpu.SemaphoreType.DMA,
            pltpu.SemaphoreType.DMA,
        ],
        compiler_params=pltpu.CompilerParams(use_tc_tiling_on_sc=False),
    )
    def k(actions_hbm, nbr_hbm, emb_hbm, nbrs_out, aemb_out,
          act_v, nbrs_v, aemb_v, sem1, sem2):
        wid = lax.axis_index("s") * _NC + lax.axis_index("c")
        base = wid * _APW
        pltpu.sync_copy(actions_hbm.at[pl.ds(base, _APW)], act_v)
        cp1 = pltpu.async_copy(nbr_hbm.at[act_v], nbrs_v, sem1)
        cp2 = pltpu.async_copy(emb_hbm.at[act_v], aemb_v, sem2)
        cp1.wait()
        pltpu.sync_copy(nbrs_v, nbrs_out.at[pl.ds(base, _APW), :])
        cp2.wait()
        pltpu.sync_copy(aemb_v, aemb_out.at[pl.ds(base, _APW), :])

    return k(actions, nbr, emb)


def _sc_gather_rows(idx2d, emb):
    mesh = plsc.VectorSubcoreMesh(
        core_axis_name="c", subcore_axis_name="s",
        num_cores=_NC, num_subcores=_NS)

    @functools.partial(
        pl.kernel,
        out_type=jax.ShapeDtypeStruct((_ROWS, _DF), jnp.float32),
        mesh=mesh,
        scratch_types=[
            pltpu.VMEM((_NCH, _CH), jnp.int32),
            pltpu.VMEM((_CH, _DF), jnp.float32),
            pltpu.VMEM((_CH, _DF), jnp.float32),
            pltpu.SemaphoreType.DMA,
            pltpu.SemaphoreType.DMA,
        ],
    )
    def k(idx_hbm, emb_hbm, out_hbm, idx_v, buf0, buf1, sem0, sem1):
        wid = lax.axis_index("s") * _NC + lax.axis_index("c")
        base = wid * _RPW
        pltpu.sync_copy(idx_hbm.at[pl.ds(wid * _NCH, _NCH), :], idx_v)
        bufs = (buf0, buf1)
        sems = (sem0, sem1)
        for b in range(2):
            pltpu.async_copy(emb_hbm.at[idx_v.at[b]], bufs[b], sems[b])

        def body(t, _):
            for b in range(2):
                chunk = 2 * t + b
                pltpu.make_async_copy(
                    emb_hbm.at[idx_v.at[chunk]], bufs[b], sems[b]).wait()
                pltpu.sync_copy(
                    bufs[b], out_hbm.at[pl.ds(base + chunk * _CH, _CH), :])
                pltpu.async_copy(
                    emb_hbm.at[idx_v.at[chunk + 2]], bufs[b], sems[b])
            return 0

        lax.fori_loop(0, _NCH // 2 - 1, body, 0)
        for b in range(2):
            chunk = _NCH - 2 + b
            pltpu.make_async_copy(
                emb_hbm.at[idx_v.at[chunk]], bufs[b], sems[b]).wait()
            pltpu.sync_copy(
                bufs[b], out_hbm.at[pl.ds(base + chunk * _CH, _CH), :])

    return k(idx2d, emb)


def _dense_body(step_ref, aemb_ref, h0_ref, c0_ref, ne_ref,
                A1_ref, s1_ref, b10_ref, A1b_ref, b11_ref,
                A2_ref, s2_ref, b20_ref, W21_ref, b21c_ref,
                A3_ref, b30_ref, Wih_ref, Whh_ref, bihh_ref,
                vals_ref, h1_ref, c1_ref):
    f32 = jnp.float32
    step = step_ref[...] / _DELAYED
    aemb = aemb_ref[...]
    eh = jnp.maximum(
        jnp.dot(aemb, A1_ref[...], preferred_element_type=f32)
        + step * s1_ref[...] + b10_ref[...], 0.0)
    e = jnp.dot(eh, A1b_ref[...], preferred_element_type=f32) + b11_ref[...]
    gates = (jnp.dot(e, Wih_ref[...], preferred_element_type=f32)
             + jnp.dot(h0_ref[...], Whh_ref[...], preferred_element_type=f32)
             + bihh_ref[...])
    i_g = gates[:, 0 * _H:1 * _H]
    f_g = gates[:, 1 * _H:2 * _H]
    g_g = gates[:, 2 * _H:3 * _H]
    o_g = gates[:, 3 * _H:4 * _H]
    c1 = jax.nn.sigmoid(f_g) * c0_ref[...] + jax.nn.sigmoid(i_g) * jnp.tanh(g_g)
    h1 = jax.nn.sigmoid(o_g) * jnp.tanh(c1)
    c1_ref[...] = c1
    h1_ref[...] = h1
    a = jnp.dot(h1, A3_ref[...], preferred_element_type=f32) + b30_ref[...]
    v = jnp.dot(a, W21_ref[...], preferred_element_type=f32)
    const = jnp.dot(a, b21c_ref[...], preferred_element_type=f32)
    z = jnp.dot(ne_ref[...], A2_ref[...], preferred_element_type=f32)
    z3 = (z.reshape(_BB, _DEG, 2 * _DF)
          + step[:, :, None] * s2_ref[...][None]
          + b20_ref[...][None])
    r3 = jnp.maximum(z3, 0.0)
    vals = (jnp.sum(r3 * v[:, None, :], axis=-1) + const) * np.float32(
        1.0 / np.sqrt(128.0))
    vals_ref[...] = vals


def _tc_dense(step2, a_emb, h0, c0, ne,
              A1, s1, b10, A1b, b11, A2, s2, b20, W21, b21c,
              A3, b30, Wih, Whh, bihh):
    full = lambda shape: pl.BlockSpec(shape, lambda i: (0, 0))
    grid_spec = pl.GridSpec(
        grid=(_GRID,),
        in_specs=[
            pl.BlockSpec((_BB, 1), lambda i: (i, 0)),
            pl.BlockSpec((_BB, _DF), lambda i: (i, 0)),
            pl.BlockSpec((_BB, _H), lambda i: (i, 0)),
            pl.BlockSpec((_BB, _H), lambda i: (i, 0)),
            pl.BlockSpec((_BB * _DEG, _DF), lambda i: (i, 0)),
            full(A1.shape), full(s1.shape), full(b10.shape),
            full(A1b.shape), full(b11.shape),
            full(A2.shape), full(s2.shape), full(b20.shape),
            full(W21.shape), full(b21c.shape),
            full(A3.shape), full(b30.shape),
            full(Wih.shape), full(Whh.shape), full(bihh.shape),
        ],
        out_specs=[
            pl.BlockSpec((_BB, _DEG), lambda i: (i, 0)),
            pl.BlockSpec((_BB, _H), lambda i: (i, 0)),
            pl.BlockSpec((_BB, _H), lambda i: (i, 0)),
        ],
    )
    return pl.pallas_call(
        _dense_body,
        grid_spec=grid_spec,
        out_shape=[
            jax.ShapeDtypeStruct((_B, _DEG), jnp.float32),
            jax.ShapeDtypeStruct((_B, _H), jnp.float32),
            jax.ShapeDtypeStruct((_B, _H), jnp.float32),
        ],
        compiler_params=pltpu.CompilerParams(
            dimension_semantics=("arbitrary",)),
    )(step2, a_emb, h0, c0, ne,
      A1, s1, b10, A1b, b11, A2, s2, b20, W21, b21c,
      A3, b30, Wih, Whh, bihh)


def kernel(actions0, step_embs, h0, c0, nbr, node_pre_embedding,
           W1_0, b1_0, W1_1, b1_1, W2_0, b2_0, W2_1, b2_1,
           W3_0, b3_0, W_ih, W_hh, b_ih, b_hh):
    actions0 = actions0.astype(jnp.int32)
    nbr = nbr.astype(jnp.int32)

    neighbors, a_emb = _sc_gather_actions(actions0, nbr, node_pre_embedding)
    idx2d = neighbors.reshape(_ROWS // _CH, _CH)
    nbr_emb = _sc_gather_rows(idx2d, node_pre_embedding)

    A1 = W1_0[:, :_DF].T
    s1 = W1_0[:, _DF].reshape(1, 2 * _DF)
    b10 = b1_0.reshape(1, 2 * _DF)
    A1b = W1_1.T
    b11 = b1_1.reshape(1, _DF)
    A2 = W2_0[:, :_DF].T
    s2 = W2_0[:, _DF].reshape(1, 2 * _DF)
    b20 = b2_0.reshape(1, 2 * _DF)
    b21c = b2_1.reshape(_DF, 1)
    A3 = W3_0.T
    b30 = b3_0.reshape(1, _DF)
    Wih = W_ih.T
    Whh = W_hh.T
    bihh = (b_ih + b_hh).reshape(1, 4 * _H)
    step2 = step_embs.reshape(_B, 1)

    values, h1, c1 = _tc_dense(
        step2, a_emb, h0, c0, nbr_emb,
        A1, s1, b10, A1b, b11, A2, s2, b20, W2_1, b21c,
        A3, b30, Wih, Whh, bihh)
    return (neighbors, values, h1, c1)


# speedup vs baseline: 3.9447x; 1.5141x over previous
import functools

import jax
import jax.numpy as jnp
import numpy as np
from jax import lax
from jax.experimental import pallas as pl
from jax.experimental.pallas import tpu as pltpu
from jax.experimental.pallas import tpu_sc as plsc

_N = 50000
_DEG = 32
_DF = 128
_B = 4096
_H = 256
_DELAYED = 100.0

_NC = 2
_NS = 16
_NW = _NC * _NS
_APW = _B // _NW
_ROWS = _B * _DEG
_RPW = _ROWS // _NW
_CH = 128
_NCH = _RPW // _CH

_BBA = 1024
_BB = 128
_GRID = _B // _BB


def _sc_gather_nbrs(actions, nbr):
    mesh = plsc.VectorSubcoreMesh(
        core_axis_name="c", subcore_axis_name="s",
        num_cores=_NC, num_subcores=_NS)

    @functools.partial(
        pl.kernel,
        out_type=jax.ShapeDtypeStruct((_B, _DEG), jnp.int32),
        mesh=mesh,
        scratch_types=[
            pltpu.VMEM((_APW,), jnp.int32),
            pltpu.VMEM((_APW, _DEG), jnp.int32),
            pltpu.SemaphoreType.DMA,
        ],
        compiler_params=pltpu.CompilerParams(use_tc_tiling_on_sc=False),
    )
    def k(actions_hbm, nbr_hbm, nbrs_out, act_v, nbrs_v, sem):
        wid = lax.axis_index("s") * _NC + lax.axis_index("c")
        base = wid * _APW
        pltpu.sync_copy(actions_hbm.at[pl.ds(base, _APW)], act_v)
        pltpu.async_copy(nbr_hbm.at[act_v], nbrs_v, sem).wait()
        pltpu.sync_copy(nbrs_v, nbrs_out.at[pl.ds(base, _APW), :])

    return k(actions, nbr)


def _sc_gather_aemb(actions, emb):
    mesh = plsc.VectorSubcoreMesh(
        core_axis_name="c", subcore_axis_name="s",
        num_cores=_NC, num_subcores=_NS)

    @functools.partial(
        pl.kernel,
        out_type=jax.ShapeDtypeStruct((_B, _DF), jnp.float32),
        mesh=mesh,
        scratch_types=[
            pltpu.VMEM((_APW,), jnp.int32),
            pltpu.VMEM((_APW, _DF), jnp.float32),
            pltpu.SemaphoreType.DMA,
        ],
    )
    def k(actions_hbm, emb_hbm, aemb_out, act_v, aemb_v, sem):
        wid = lax.axis_index("s") * _NC + lax.axis_index("c")
        base = wid * _APW
        pltpu.sync_copy(actions_hbm.at[pl.ds(base, _APW)], act_v)
        pltpu.async_copy(emb_hbm.at[act_v], aemb_v, sem).wait()
        pltpu.sync_copy(aemb_v, aemb_out.at[pl.ds(base, _APW), :])

    return k(actions, emb)


def _sc_gather_rows(idx2d, emb):
    mesh = plsc.VectorSubcoreMesh(
        core_axis_name="c", subcore_axis_name="s",
        num_cores=_NC, num_subcores=_NS)

    @functools.partial(
        pl.kernel,
        out_type=jax.ShapeDtypeStruct((_ROWS, _DF), jnp.float32),
        mesh=mesh,
        scratch_types=[
            pltpu.VMEM((_NCH, _CH), jnp.int32),
            pltpu.VMEM((_CH, _DF), jnp.float32),
            pltpu.VMEM((_CH, _DF), jnp.float32),
            pltpu.SemaphoreType.DMA,
            pltpu.SemaphoreType.DMA,
        ],
    )
    def k(idx_hbm, emb_hbm, out_hbm, idx_v, buf0, buf1, sem0, sem1):
        wid = lax.axis_index("s") * _NC + lax.axis_index("c")
        base = wid * _RPW
        pltpu.sync_copy(idx_hbm.at[pl.ds(wid * _NCH, _NCH), :], idx_v)
        bufs = (buf0, buf1)
        sems = (sem0, sem1)
        for b in range(2):
            pltpu.async_copy(emb_hbm.at[idx_v.at[b]], bufs[b], sems[b])

        def body(t, _):
            for b in range(2):
                chunk = 2 * t + b
                pltpu.make_async_copy(
                    emb_hbm.at[idx_v.at[chunk]], bufs[b], sems[b]).wait()
                pltpu.sync_copy(
                    bufs[b], out_hbm.at[pl.ds(base + chunk * _CH, _CH), :])
                pltpu.async_copy(
                    emb_hbm.at[idx_v.at[chunk + 2]], bufs[b], sems[b])
            return 0

        lax.fori_loop(0, _NCH // 2 - 1, body, 0)
        for b in range(2):
            chunk = _NCH - 2 + b
            pltpu.make_async_copy(
                emb_hbm.at[idx_v.at[chunk]], bufs[b], sems[b]).wait()
            pltpu.sync_copy(
                bufs[b], out_hbm.at[pl.ds(base + chunk * _CH, _CH), :])

    return k(idx2d, emb)


def _action_body(step_ref, aemb_ref, h0_ref, c0_ref,
                 A1_ref, s1_ref, b10_ref, A1b_ref, b11_ref,
                 W21_ref, b21c_ref, A3_ref, b30_ref,
                 Wih_ref, Whh_ref, bihh_ref,
                 h1_ref, c1_ref, v_ref, const_ref):
    f32 = jnp.float32
    step = step_ref[...] / _DELAYED
    aemb = aemb_ref[...]
    eh = jnp.maximum(
        jnp.dot(aemb, A1_ref[...], preferred_element_type=f32)
        + step * s1_ref[...] + b10_ref[...], 0.0)
    e = jnp.dot(eh, A1b_ref[...], preferred_element_type=f32) + b11_ref[...]
    gates = (jnp.dot(e, Wih_ref[...], preferred_element_type=f32)
             + jnp.dot(h0_ref[...], Whh_ref[...], preferred_element_type=f32)
             + bihh_ref[...])
    i_g = gates[:, 0 * _H:1 * _H]
    f_g = gates[:, 1 * _H:2 * _H]
    g_g = gates[:, 2 * _H:3 * _H]
    o_g = gates[:, 3 * _H:4 * _H]
    c1 = jax.nn.sigmoid(f_g) * c0_ref[...] + jax.nn.sigmoid(i_g) * jnp.tanh(g_g)
    h1 = jax.nn.sigmoid(o_g) * jnp.tanh(c1)
    c1_ref[...] = c1
    h1_ref[...] = h1
    a = jnp.dot(h1, A3_ref[...], preferred_element_type=f32) + b30_ref[...]
    inv = np.float32(1.0 / np.sqrt(128.0))
    v_ref[...] = jnp.dot(a, W21_ref[...], preferred_element_type=f32) * inv
    const_ref[...] = jnp.dot(a, b21c_ref[...], preferred_element_type=f32) * inv


def _tc_action(step2, a_emb, h0, c0,
               A1, s1, b10, A1b, b11, W21, b21c, A3, b30, Wih, Whh, bihh):
    full = lambda shape: pl.BlockSpec(shape, lambda i: (0, 0))
    grid_spec = pl.GridSpec(
        grid=(_B // _BBA,),
        in_specs=[
            pl.BlockSpec((_BBA, 1), lambda i: (i, 0)),
            pl.BlockSpec((_BBA, _DF), lambda i: (i, 0)),
            pl.BlockSpec((_BBA, _H), lambda i: (i, 0)),
            pl.BlockSpec((_BBA, _H), lambda i: (i, 0)),
            full(A1.shape), full(s1.shape), full(b10.shape),
            full(A1b.shape), full(b11.shape),
            full(W21.shape), full(b21c.shape),
            full(A3.shape), full(b30.shape),
            full(Wih.shape), full(Whh.shape), full(bihh.shape),
        ],
        out_specs=[
            pl.BlockSpec((_BBA, _H), lambda i: (i, 0)),
            pl.BlockSpec((_BBA, _H), lambda i: (i, 0)),
            pl.BlockSpec((_BBA, 2 * _DF), lambda i: (i, 0)),
            pl.BlockSpec((_BBA, 1), lambda i: (i, 0)),
        ],
    )
    return pl.pallas_call(
        _action_body,
        grid_spec=grid_spec,
        out_shape=[
            jax.ShapeDtypeStruct((_B, _H), jnp.float32),
            jax.ShapeDtypeStruct((_B, _H), jnp.float32),
            jax.ShapeDtypeStruct((_B, 2 * _DF), jnp.float32),
            jax.ShapeDtypeStruct((_B, 1), jnp.float32),
        ],
        compiler_params=pltpu.CompilerParams(
            dimension_semantics=("arbitrary",)),
    )(step2, a_emb, h0, c0,
      A1, s1, b10, A1b, b11, W21, b21c, A3, b30, Wih, Whh, bihh)


def _nbr_body(step_ref, ne_ref, v_ref, const_ref,
              A2_ref, s2_ref, b20_ref, ones_ref, vals_ref):
    f32 = jnp.float32
    step = step_ref[...] / _DELAYED
    z = jnp.dot(ne_ref[...], A2_ref[...], preferred_element_type=f32)
    z3 = (z.reshape(_BB, _DEG, 2 * _DF)
          + step[:, :, None] * s2_ref[...][None]
          + b20_ref[...][None])
    r3 = jnp.maximum(z3, 0.0)
    R = (r3 * v_ref[...][:, None, :]).reshape(_BB * _DEG, 2 * _DF)
    col = jnp.dot(R, ones_ref[...], preferred_element_type=f32)
    vals_ref[...] = col.reshape(_BB, _DEG) + const_ref[...]


def _tc_nbr(step2, ne, v, const, A2, s2, b20):
    ones_col = jnp.ones((2 * _DF, 1), jnp.float32)
    full = lambda shape: pl.BlockSpec(shape, lambda i: (0, 0))
    grid_spec = pl.GridSpec(
        grid=(_GRID,),
        in_specs=[
            pl.BlockSpec((_BB, 1), lambda i: (i, 0)),
            pl.BlockSpec((_BB * _DEG, _DF), lambda i: (i, 0)),
            pl.BlockSpec((_BB, 2 * _DF), lambda i: (i, 0)),
            pl.BlockSpec((_BB, 1), lambda i: (i, 0)),
            full(A2.shape), full(s2.shape), full(b20.shape),
            full(ones_col.shape),
        ],
        out_specs=[
            pl.BlockSpec((_BB, _DEG), lambda i: (i, 0)),
        ],
    )
    return pl.pallas_call(
        _nbr_body,
        grid_spec=grid_spec,
        out_shape=[
            jax.ShapeDtypeStruct((_B, _DEG), jnp.float32),
        ],
        compiler_params=pltpu.CompilerParams(
            dimension_semantics=("arbitrary",)),
    )(step2, ne, v, const, A2, s2, b20, ones_col)[0]


def kernel(actions0, step_embs, h0, c0, nbr, node_pre_embedding,
           W1_0, b1_0, W1_1, b1_1, W2_0, b2_0, W2_1, b2_1,
           W3_0, b3_0, W_ih, W_hh, b_ih, b_hh):
    actions0 = actions0.astype(jnp.int32)
    nbr = nbr.astype(jnp.int32)

    neighbors = _sc_gather_nbrs(actions0, nbr)
    a_emb = _sc_gather_aemb(actions0, node_pre_embedding)
    idx2d = neighbors.reshape(_ROWS // _CH, _CH)
    nbr_emb = _sc_gather_rows(idx2d, node_pre_embedding)

    A1 = W1_0[:, :_DF].T
    s1 = W1_0[:, _DF].reshape(1, 2 * _DF)
    b10 = b1_0.reshape(1, 2 * _DF)
    A1b = W1_1.T
    b11 = b1_1.reshape(1, _DF)
    A2 = W2_0[:, :_DF].T
    s2 = W2_0[:, _DF].reshape(1, 2 * _DF)
    b20 = b2_0.reshape(1, 2 * _DF)
    b21c = b2_1.reshape(_DF, 1)
    A3 = W3_0.T
    b30 = b3_0.reshape(1, _DF)
    Wih = W_ih.T
    Whh = W_hh.T
    bihh = (b_ih + b_hh).reshape(1, 4 * _H)
    step2 = step_embs.reshape(_B, 1)

    h1, c1, v, const = _tc_action(
        step2, a_emb, h0, c0,
        A1, s1, b10, A1b, b11, W2_1, b21c, A3, b30, Wih, Whh, bihh)
    values = _tc_nbr(step2, nbr_emb, v, const, A2, s2, b20)
    return (neighbors, values, h1, c1)
